# Initial kernel scaffold; baseline (speedup 1.0000x reference)
import jax
import jax.numpy as jnp
from jax.experimental import pallas as pl


def kernel(seq, adj, W, a):
    raise NotImplementedError("write your pallas kernel here")


# speedup vs baseline: 1.0078x; 1.0078x over previous
import jax
import jax.numpy as jnp
from jax.experimental import pallas as pl
from jax.experimental.pallas import tpu as pltpu


def _gcn_block(seq_ref, adj_ref, w_ref, a_ref, out_ref):
    h = jnp.dot(adj_ref[...], seq_ref[...],
                preferred_element_type=jnp.float32)
    y = jax.lax.dot_general(h, w_ref[...], (((1,), (1,)), ((), ())),
                            preferred_element_type=jnp.float32)
    slope = a_ref[0, 0]
    out_ref[...] = jnp.where(y >= 0, y, slope * y)


def kernel(seq, adj, W, a):
    N, d_in = seq.shape
    d_out = W.shape[0]
    BM = 400
    grid = (N // BM,)
    return pl.pallas_call(
        _gcn_block,
        grid=grid,
        in_specs=[
            pl.BlockSpec((N, d_in), lambda i: (0, 0)),
            pl.BlockSpec((BM, N), lambda i: (i, 0)),
            pl.BlockSpec((d_out, d_in), lambda i: (0, 0)),
            pl.BlockSpec(memory_space=pltpu.SMEM),
        ],
        out_specs=pl.BlockSpec((BM, d_out), lambda i: (i, 0)),
        out_shape=jax.ShapeDtypeStruct((N, d_out), jnp.float32),
    )(seq, adj, W, a.reshape(1, 1))
